# baseline (speedup 1.0000x reference)
import functools

import jax
import jax.numpy as jnp
from jax import lax
from jax.experimental import pallas as pl
from jax.experimental.pallas import tpu as pltpu
from jax.experimental.pallas import tpu_sc as plsc

N = 10000
E = 640000
D = 128
ED = 16
H = 64
L = 4
G = 64

NC = 2
NS = 16
NW = NC * NS
NB = 320
NBK = 32
ROWS = NBK * NB
CAP = 20480
OCAP = 2560
SS = NBK * CAP + OCAP
C = 80
EWB = E // NS
EWS = SS // NW
FWB = SS // NS
BE = 2560
_f32 = jnp.float32
_i32 = jnp.int32

_SENT = -1.0


def _sc_bin_body(pos_hbm, rec_hbm, out_hbm, posb, rbuf, sbuf, sem):
  sid = lax.axis_index("s")

  def srow(r, carry):
    for j in range(8):
      sbuf[r, pl.ds(j * 16, 16)] = jnp.full((16,), _SENT, _f32)
    return carry

  lax.fori_loop(0, C, srow, 0)

  def fill(i, carry):
    pltpu.sync_copy(sbuf, out_hbm.at[pl.ds(sid * FWB + i * C, C)])
    return carry

  lax.fori_loop(0, FWB // C, fill, 0)
  plsc.subcore_barrier()

  def chunk(i, carry):
    off = sid * EWB + i * C
    pltpu.sync_copy(pos_hbm.at[pl.ds(off, C)], posb)
    pltpu.async_copy(rec_hbm.at[pl.ds(off, C)], rbuf, sem).wait()
    pltpu.sync_copy(rbuf, out_hbm.at[posb])
    return carry

  lax.fori_loop(0, EWB // C, chunk, 0)


def _sc_gather_body(dst_hbm, src_hbm, td_hbm, ts_hbm, g_hbm,
                    idxd, idxs, bufd, bufs, semd, sems):
  wid = lax.axis_index("s") * NC + lax.axis_index("c")
  base = wid * EWS

  def chunk(i, carry):
    off = base + i * C
    pltpu.sync_copy(dst_hbm.at[pl.ds(off, C)], idxd)
    pltpu.sync_copy(src_hbm.at[pl.ds(off, C)], idxs)
    cpd = pltpu.async_copy(td_hbm.at[idxd], bufd, semd)
    cps = pltpu.async_copy(ts_hbm.at[idxs], bufs, sems)
    cpd.wait()
    cps.wait()

    def row(r, c2):
      for j in range(8):
        s = pl.ds(j * 16, 16)
        bufd[r, s] = bufd[r, s] + bufs[r, s]
      return c2

    lax.fori_loop(0, C, row, 0)
    pltpu.sync_copy(bufd, g_hbm.at[pl.ds(off, C)])
    return carry

  lax.fori_loop(0, EWS // C, chunk, 0)


@functools.cache
def _sc_kernels():
  mesh1 = plsc.VectorSubcoreMesh(core_axis_name="c", subcore_axis_name="s",
                                 num_cores=1)
  mesh2 = plsc.VectorSubcoreMesh(core_axis_name="c", subcore_axis_name="s")
  binning = functools.partial(
      pl.kernel,
      out_type=jax.ShapeDtypeStruct((SS, 128), _f32),
      mesh=mesh1,
      scratch_types=[
          pltpu.VMEM((C,), _i32),
          pltpu.VMEM((C, 128), _f32),
          pltpu.VMEM((C, 128), _f32),
          pltpu.SemaphoreType.DMA,
      ],
  )(_sc_bin_body)
  gather = functools.partial(
      pl.kernel,
      out_type=jax.ShapeDtypeStruct((SS, 2 * H), _f32),
      mesh=mesh2,
      scratch_types=[
          pltpu.VMEM((C,), _i32),
          pltpu.VMEM((C,), _i32),
          pltpu.VMEM((C, 2 * H), _f32),
          pltpu.VMEM((C, 2 * H), _f32),
          pltpu.SemaphoreType.DMA,
          pltpu.SemaphoreType.DMA,
      ],
  )(_sc_gather_body)
  return binning, gather


def _cumsum0(x):
  s = 1
  n = x.shape[0]
  while s < n:
    x = x + jnp.concatenate([jnp.zeros((s, x.shape[1]), x.dtype), x[:-s]],
                            axis=0)
    s *= 2
  return x


def _rank_body(dst_ref, posb_ref, ovf_ref, c32, cov):
  pid = pl.program_id(0)

  @pl.when(pid == 0)
  def _init():
    c32[...] = jnp.zeros((1, NBK), _f32)
    cov[...] = jnp.zeros((1, NBK), _f32)

  d = dst_ref[0, 0, :]
  b = d // NB
  oh = (b[:, None] == lax.broadcasted_iota(_i32, (BE, NBK), 1)).astype(_f32)
  csum = _cumsum0(oh)
  posb = jnp.sum((csum - oh + c32[...]) * oh, axis=1)
  ovf = (posb >= CAP).astype(_f32)
  ovfrank = _cumsum0(ovf[:, None])[:, 0] - ovf + cov[0, 0]
  posb_ref[0, 0, :] = posb.astype(_i32)
  ovf_ref[0, 0, :] = ovfrank.astype(_i32)
  c32[...] = c32[...] + jnp.sum(oh, axis=0, keepdims=True)
  cov[...] = cov[...] + jnp.sum(ovf)


def _pos_body(dst_ref, src_ref, ea_ref, posb_ref, ovf_ref, pos_ref, rec_ref):
  d = dst_ref[0, 0, :]
  b = d // NB
  posb = posb_ref[0, 0, :]
  ovfrank = jnp.minimum(ovf_ref[0, 0, :], OCAP - 1)
  pos_ref[0, 0, :] = jnp.where(posb < CAP, b * CAP + posb,
                               NBK * CAP + ovfrank)
  db = lax.bitcast_convert_type(d, _f32)
  sb = lax.bitcast_convert_type(src_ref[0, 0, :], _f32)
  rec_ref[...] = jnp.concatenate(
      [db[:, None], sb[:, None], ea_ref[...], jnp.zeros((BE, 110), _f32)],
      axis=1)


def _extract_body(rec_ref, dst_ref, src_ref, loc_ref, dstv_ref, ea_ref):
  d = lax.bitcast_convert_type(rec_ref[:, 0], _i32)
  s = lax.bitcast_convert_type(rec_ref[:, 1], _i32)
  valid = (d >= 0) & (d < N)
  dst_ref[0, 0, :] = jnp.clip(d, 0, N - 1)
  src_ref[0, 0, :] = jnp.clip(s, 0, N - 1)
  loc_ref[0, 0, :] = jnp.where(valid, d % NB, 9999)
  dstv_ref[0, 0, :] = jnp.where(valid, d, 20000)
  ea_ref[...] = rec_ref[:, 2:2 + ED]


def _segm_body(loc_ref, m_ref, out_ref):
  jb = pl.program_id(1)

  @pl.when(jb == 0)
  def _init():
    out_ref[...] = jnp.zeros((NB, H), _f32)

  ohT = (lax.broadcasted_iota(_i32, (NB, BE), 0)
         == loc_ref[0, 0, :][None, :]).astype(_f32)
  out_ref[...] = out_ref[...] + jnp.dot(ohT, m_ref[...],
                                        preferred_element_type=_f32)


def _ovf_body(dstv_ref, m_ref, out_ref):
  i = pl.program_id(0)
  ohT = ((lax.broadcasted_iota(_i32, (BE, BE), 0) + i * BE)
         == dstv_ref[0, 0, :][None, :]).astype(_f32)
  out_ref[...] = jnp.dot(ohT, m_ref[...], preferred_element_type=_f32)


def _embed_body(x_ref, we_ref, be_ref, wd_ref, ws_ref, h_ref, td_ref, ts_ref):
  h = jnp.maximum(jnp.dot(x_ref[...], we_ref[...],
                          preferred_element_type=_f32) + be_ref[...], 0.0)
  h_ref[...] = h
  td_ref[...] = jnp.dot(h, wd_ref[...], preferred_element_type=_f32)
  ts_ref[...] = jnp.dot(h, ws_ref[...], preferred_element_type=_f32)


def _edge_body(g_ref, ea_ref, we_ref, b_ref, m_ref):
  z = g_ref[...] + jnp.dot(ea_ref[...], we_ref[...],
                           preferred_element_type=_f32) + b_ref[...]
  zf = z[:, :H]
  zs = z[:, H:]
  sig = 1.0 / (1.0 + jnp.exp(-zf))
  sp = jnp.maximum(zs, 0.0) + jnp.log(1.0 + jnp.exp(-jnp.abs(zs)))
  m_ref[...] = sig * sp


def _bn_update(h, p1_ref, p2_ref, gm_ref, bt_ref):
  conv = h + p1_ref[...] + p2_ref[...]
  mu = jnp.mean(conv, axis=0, keepdims=True)
  dc = conv - mu
  var = jnp.mean(dc * dc, axis=0, keepdims=True)
  hn = jnp.maximum(dc * lax.rsqrt(var + 1e-5) * gm_ref[...] + bt_ref[...], 0.0)
  return h + hn


def _update_body(h_ref, p1_ref, p2_ref, gm_ref, bt_ref, wd_ref, ws_ref,
                 h_out, td_ref, ts_ref):
  h2 = _bn_update(h_ref[...], p1_ref, p2_ref, gm_ref, bt_ref)
  h_out[...] = h2
  td_ref[...] = jnp.dot(h2, wd_ref[...], preferred_element_type=_f32)
  ts_ref[...] = jnp.dot(h2, ws_ref[...], preferred_element_type=_f32)


def _final_body(h_ref, p1_ref, p2_ref, gm_ref, bt_ref, batch_ref,
                w1_ref, b1_ref, w2_ref, b2_ref, wbg_ref, bbg_ref, out_ref):
  h2 = _bn_update(h_ref[...], p1_ref, p2_ref, gm_ref, bt_ref)
  seg = lax.broadcasted_iota(_i32, (G, N), 0)
  oh = (batch_ref[...] == seg).astype(_f32)
  sums = jnp.dot(oh, h2, preferred_element_type=_f32)
  counts = jnp.sum(oh, axis=1, keepdims=True)
  pooled = sums / jnp.maximum(counts, 1.0)
  o = jnp.maximum(jnp.dot(pooled, w1_ref[...],
                          preferred_element_type=_f32) + b1_ref[...], 0.0)
  o = jnp.maximum(jnp.dot(o, w2_ref[...],
                          preferred_element_type=_f32) + b2_ref[...], 0.0)
  out_ref[...] = jnp.dot(o, wbg_ref[...],
                         preferred_element_type=_f32) + bbg_ref[...]


_rank_call = pl.pallas_call(
    _rank_body,
    grid=(E // BE,),
    in_specs=[pl.BlockSpec((1, 1, BE), lambda i: (i, 0, 0))],
    out_specs=[pl.BlockSpec((1, 1, BE), lambda i: (i, 0, 0)),
               pl.BlockSpec((1, 1, BE), lambda i: (i, 0, 0))],
    out_shape=[jax.ShapeDtypeStruct((E // BE, 1, BE), _i32),
               jax.ShapeDtypeStruct((E // BE, 1, BE), _i32)],
    scratch_shapes=[pltpu.VMEM((1, NBK), _f32), pltpu.VMEM((1, NBK), _f32)],
)

_pos_call = pl.pallas_call(
    _pos_body,
    grid=(E // BE,),
    in_specs=[pl.BlockSpec((1, 1, BE), lambda i: (i, 0, 0)),
              pl.BlockSpec((1, 1, BE), lambda i: (i, 0, 0)),
              pl.BlockSpec((BE, ED), lambda i: (i, 0)),
              pl.BlockSpec((1, 1, BE), lambda i: (i, 0, 0)),
              pl.BlockSpec((1, 1, BE), lambda i: (i, 0, 0))],
    out_specs=[pl.BlockSpec((1, 1, BE), lambda i: (i, 0, 0)),
               pl.BlockSpec((BE, 128), lambda i: (i, 0))],
    out_shape=[jax.ShapeDtypeStruct((E // BE, 1, BE), _i32),
               jax.ShapeDtypeStruct((E, 128), _f32)],
)

_extract_call = pl.pallas_call(
    _extract_body,
    grid=(SS // BE,),
    in_specs=[pl.BlockSpec((BE, 128), lambda i: (i, 0))],
    out_specs=[pl.BlockSpec((1, 1, BE), lambda i: (i, 0, 0)),
               pl.BlockSpec((1, 1, BE), lambda i: (i, 0, 0)),
               pl.BlockSpec((1, 1, BE), lambda i: (i, 0, 0)),
               pl.BlockSpec((1, 1, BE), lambda i: (i, 0, 0)),
               pl.BlockSpec((BE, ED), lambda i: (i, 0))],
    out_shape=[jax.ShapeDtypeStruct((SS // BE, 1, BE), _i32),
               jax.ShapeDtypeStruct((SS // BE, 1, BE), _i32),
               jax.ShapeDtypeStruct((SS // BE, 1, BE), _i32),
               jax.ShapeDtypeStruct((SS // BE, 1, BE), _i32),
               jax.ShapeDtypeStruct((SS, ED), _f32)],
)

_segm_call = pl.pallas_call(
    _segm_body,
    grid=(NBK, CAP // BE),
    in_specs=[pl.BlockSpec((1, 1, BE), lambda w, j: (w * (CAP // BE) + j, 0, 0)),
              pl.BlockSpec((BE, H), lambda w, j: (w * (CAP // BE) + j, 0))],
    out_specs=pl.BlockSpec((NB, H), lambda w, j: (w, 0)),
    out_shape=jax.ShapeDtypeStruct((ROWS, H), _f32),
)

_ovf_call = pl.pallas_call(
    _ovf_body,
    grid=(ROWS // BE,),
    in_specs=[pl.BlockSpec((1, 1, BE), lambda i: (SS // BE - 1, 0, 0)),
              pl.BlockSpec((BE, H), lambda i: (SS // BE - 1, 0))],
    out_specs=pl.BlockSpec((BE, H), lambda i: (i, 0)),
    out_shape=jax.ShapeDtypeStruct((ROWS, H), _f32),
)

_embed_call = pl.pallas_call(
    _embed_body,
    out_shape=[jax.ShapeDtypeStruct((N, H), _f32),
               jax.ShapeDtypeStruct((N, 2 * H), _f32),
               jax.ShapeDtypeStruct((N, 2 * H), _f32)],
)

_edge_call = pl.pallas_call(
    _edge_body,
    grid=(SS // BE,),
    in_specs=[pl.BlockSpec((BE, 2 * H), lambda i: (i, 0)),
              pl.BlockSpec((BE, ED), lambda i: (i, 0)),
              pl.BlockSpec((ED, 2 * H), lambda i: (0, 0)),
              pl.BlockSpec((1, 2 * H), lambda i: (0, 0))],
    out_specs=pl.BlockSpec((BE, H), lambda i: (i, 0)),
    out_shape=jax.ShapeDtypeStruct((SS, H), _f32),
)

_update_call = pl.pallas_call(
    _update_body,
    out_shape=[jax.ShapeDtypeStruct((N, H), _f32),
               jax.ShapeDtypeStruct((N, 2 * H), _f32),
               jax.ShapeDtypeStruct((N, 2 * H), _f32)],
)

_final_call = pl.pallas_call(
    _final_body,
    out_shape=jax.ShapeDtypeStruct((G, 128), _f32),
)


def kernel(x, edge_index, edge_attr, batch, W_emb, b_emb, Wf, bf, Ws, bs,
           gamma, beta, W1, b1, W2, b2, Wbg, bbg):
  src = edge_index[0]
  dst = edge_index[1]

  Wd = jnp.concatenate([Wf[:, :H, :], Ws[:, :H, :]], axis=2)
  Wsr = jnp.concatenate([Wf[:, H:2 * H, :], Ws[:, H:2 * H, :]], axis=2)
  We = jnp.concatenate([Wf[:, 2 * H:, :], Ws[:, 2 * H:, :]], axis=2)
  bc = jnp.concatenate([bf, bs], axis=1)
  wbg_pad = jnp.zeros((32, 128), _f32).at[:, 0].set(Wbg[:, 0])
  bbg_pad = jnp.zeros((1, 128), _f32).at[0, 0].set(bbg[0])

  sc_bin, sc_gather = _sc_kernels()

  dst3 = dst.reshape(E // BE, 1, BE)
  src3 = src.reshape(E // BE, 1, BE)
  posb3, ovf3 = _rank_call(dst3)
  pos3, rec = _pos_call(dst3, src3, edge_attr, posb3, ovf3)
  rec_s = sc_bin(pos3.reshape(E), rec)
  dst3s, src3s, loc3s, dstv3s, ea_s = _extract_call(rec_s)
  dst_s = dst3s.reshape(SS)
  src_s = src3s.reshape(SS)

  h, td, ts = _embed_call(x, W_emb, b_emb.reshape(1, H), Wd[0], Wsr[0])
  out = None
  for l in range(L):
    g = sc_gather(dst_s, src_s, td, ts)
    m = _edge_call(g, ea_s, We[l], bc[l].reshape(1, 2 * H))
    agg1 = _segm_call(loc3s, m)[:N]
    agg2 = _ovf_call(dstv3s, m)[:N]
    if l < L - 1:
      h, td, ts = _update_call(h, agg1, agg2, gamma[l].reshape(1, H),
                               beta[l].reshape(1, H), Wd[l + 1], Wsr[l + 1])
    else:
      out = _final_call(h, agg1, agg2, gamma[l].reshape(1, H),
                        beta[l].reshape(1, H), batch.reshape(1, N),
                        W1, b1.reshape(1, 64), W2, b2.reshape(1, 32),
                        wbg_pad, bbg_pad)
  return out[:, 0]
